# baseline (speedup 1.0000x reference)
import functools

import jax
import jax.numpy as jnp
from jax import lax
from jax.experimental import pallas as pl
from jax.experimental.pallas import tpu as pltpu
from jax.experimental.pallas import tpu_sc as plsc

NW = 32
L = 16
N_NEG = 64
NSLOT = 8


def _make_sc_call(B, D, V):
    BW = B // NW
    NCTX = BW // N_NEG
    mesh = plsc.VectorSubcoreMesh(core_axis_name="c", subcore_axis_name="s")

    @functools.partial(
        pl.kernel,
        out_type=[
            jax.ShapeDtypeStruct((NW, BW), jnp.float32),
            jax.ShapeDtypeStruct((NW, BW, N_NEG), jnp.float32),
        ],
        mesh=mesh,
        compiler_params=pltpu.CompilerParams(
            needs_layout_passes=False, use_tc_tiling_on_sc=False),
        scratch_types=[
            pltpu.VMEM((NCTX, N_NEG), jnp.int32),
            pltpu.VMEM((BW, N_NEG), jnp.int32),
            pltpu.VMEM((BW, D), jnp.float32),
            pltpu.VMEM((BW,), jnp.float32),
            pltpu.VMEM((BW, N_NEG), jnp.float32),
        ]
        + [pltpu.VMEM((N_NEG, D), jnp.bfloat16)] * NSLOT
        + [pltpu.SemaphoreType.DMA] * NSLOT,
    )
    def sc_call(table, ctx, negs, emb, out_t, out_n,
                idxc_v, idxn_v, emb_v, outt_v, outn_v, *ring):
        bufs = ring[:NSLOT]
        sems = ring[NSLOT:]
        wid = lax.axis_index("s") * 2 + lax.axis_index("c")
        lanes = lax.iota(jnp.int32, L)
        zeros = jnp.zeros((L,), jnp.float32)

        pltpu.sync_copy(ctx.at[wid], idxc_v)
        pltpu.sync_copy(negs.at[wid], idxn_v)
        pltpu.sync_copy(emb.at[wid], emb_v)

        def fire_neg(b, s):
            pltpu.make_async_copy(
                table.at[idxn_v.at[b]], bufs[s], sems[s]).start()

        def wait(s):
            pltpu.make_async_copy(
                table.at[idxn_v.at[0]], bufs[s], sems[s]).wait()

        def row_dot(rows_v, r, e0, e1, e2, e3):
            lo = rows_v[r, pl.ds(0, 2 * L)]
            hi = rows_v[r, pl.ds(2 * L, 2 * L)]
            a0, a1 = plsc.unpack(
                lo, format=plsc.PackFormat.INTERLEAVED,
                preferred_element_type=jnp.float32)
            b0, b1 = plsc.unpack(
                hi, format=plsc.PackFormat.INTERLEAVED,
                preferred_element_type=jnp.float32)
            return (a0 * e0 + a1 * e1) + (b0 * e2 + b1 * e3)

        def compute_row(b, rows_v):
            e0 = emb_v[b, pl.ds(0, L)]
            e1 = emb_v[b, pl.ds(L, L)]
            e2 = emb_v[b, pl.ds(2 * L, L)]
            e3 = emb_v[b, pl.ds(3 * L, L)]
            for g in range(4):
                res = zeros
                for n in range(L):
                    acc = row_dot(rows_v, g * L + n, e0, e1, e2, e3)
                    res = jnp.where(lanes == n, jnp.sum(acc), res)
                outn_v[b, pl.ds(g * L, L)] = -res

        for s in range(NSLOT):
            fire_neg(s, s)

        def neg_body(jj, carry):
            for s in range(NSLOT):
                b = NSLOT * jj + s
                wait(s)
                compute_row(b, bufs[s])

                @pl.when(b + NSLOT < BW)
                def _():
                    fire_neg(b + NSLOT, s)

            return carry

        lax.fori_loop(0, BW // NSLOT, neg_body, 0)

        def fire_tgt(t, s):
            pltpu.make_async_copy(
                table.at[idxc_v.at[t]], bufs[s], sems[s]).start()

        for s in range(NSLOT):
            fire_tgt(s, s)

        def tgt_body(tt, carry):
            for s in range(NSLOT):
                t = NSLOT * tt + s
                wait(s)
                rows_v = bufs[s]
                for g in range(4):
                    res = zeros
                    for n in range(L):
                        i = g * L + n
                        b = t * N_NEG + i
                        acc = row_dot(
                            rows_v, i,
                            emb_v[b, pl.ds(0, L)],
                            emb_v[b, pl.ds(L, L)],
                            emb_v[b, pl.ds(2 * L, L)],
                            emb_v[b, pl.ds(3 * L, L)],
                        )
                        res = jnp.where(lanes == n, jnp.sum(acc), res)
                    outt_v[pl.ds(t * N_NEG + g * L, L)] = res

                @pl.when(t + NSLOT < NCTX)
                def _():
                    fire_tgt(t + NSLOT, s)

            return carry

        lax.fori_loop(0, NCTX // NSLOT, tgt_body, 0)

        pltpu.sync_copy(outt_v, out_t.at[wid])
        pltpu.sync_copy(outn_v, out_n.at[wid])

    return sc_call


def kernel(embedded_poi_in, context, num_sampled, poi_table):
    B, D = embedded_poi_in.shape
    V = poi_table.shape[0]
    BW = B // NW

    neg_key = jax.random.fold_in(jax.random.key(0), 12345)
    negs = jax.random.randint(neg_key, (B, N_NEG), 1, V, dtype=jnp.int32)
    negs = negs + (jnp.asarray(num_sampled, jnp.int32) - jnp.int32(N_NEG))

    table_bf = poi_table.astype(jnp.bfloat16)

    cols = jnp.arange(D)
    perm = jnp.concatenate([
        cols[0:2 * L:2], cols[1:2 * L:2],
        cols[2 * L::2], cols[2 * L + 1::2],
    ])
    emb_perm = embedded_poi_in[:, perm]

    ctx = context.astype(jnp.int32).reshape(NW, BW // N_NEG, N_NEG)
    negs_r = negs.reshape(NW, BW, N_NEG)
    emb_r = emb_perm.reshape(NW, BW, D)

    out_t, out_n = _make_sc_call(B, D, V)(table_bf, ctx, negs_r, emb_r)
    return (out_t.reshape(B), out_n.reshape(B, N_NEG, 1))
